# Initial kernel scaffold; baseline (speedup 1.0000x reference)
import jax
import jax.numpy as jnp
from jax.experimental import pallas as pl


def kernel(x, table):
    raise NotImplementedError("write your pallas kernel here")


# speedup vs baseline: 1.7492x; 1.7492x over previous
import functools

import jax
import jax.numpy as jnp
from jax import lax
from jax.experimental import pallas as pl
from jax.experimental.pallas import tpu as pltpu
from jax.experimental.pallas import tpu_sc as plsc

BATCH = 16384
HIST = 50
EMBED_DIM = 64
N = BATCH * HIST
NC, NS = 2, 16
NW = NC * NS
PER_W = N // NW
CHUNK = 128
NCHUNK = PER_W // CHUNK


def _make_sc_gather():
    mesh = plsc.VectorSubcoreMesh(
        core_axis_name="c", subcore_axis_name="s", num_cores=NC, num_subcores=NS
    )

    @functools.partial(
        pl.kernel,
        out_type=jax.ShapeDtypeStruct((N, EMBED_DIM), jnp.float32),
        mesh=mesh,
        compiler_params=pltpu.CompilerParams(use_tc_tiling_on_sc=False),
        scratch_types=[
            pltpu.VMEM((NCHUNK, CHUNK), jnp.int32),
            pltpu.VMEM((CHUNK, EMBED_DIM), jnp.float32),
            pltpu.VMEM((CHUNK, EMBED_DIM), jnp.float32),
            pltpu.SemaphoreType.DMA,
            pltpu.SemaphoreType.DMA,
        ],
    )
    def k(table_hbm, idx_hbm, out_hbm, idx_v, rows0, rows1, gsem, ssem):
        wid = lax.axis_index("s") * NC + lax.axis_index("c")
        base = wid * PER_W
        pltpu.sync_copy(idx_hbm.at[wid], idx_v)

        def gather(j, buf):
            return pltpu.async_copy(table_hbm.at[idx_v.at[j]], buf, gsem)

        def put(j, buf):
            pltpu.async_copy(buf, out_hbm.at[pl.ds(base + j * CHUNK, CHUNK)], ssem)

        def drain_put(buf):
            pltpu.make_async_copy(buf, out_hbm.at[pl.ds(base, CHUNK)], ssem).wait()

        gather(0, rows0).wait()
        put(0, rows0)
        gather(1, rows1).wait()
        put(1, rows1)

        def body(i, carry):
            for b, buf in ((0, rows0), (1, rows1)):
                j = 2 * i + 2 + b
                drain_put(buf)
                gather(j, buf).wait()
                put(j, buf)
            return carry

        lax.fori_loop(0, (NCHUNK - 2) // 2, body, 0)
        drain_put(rows0)
        drain_put(rows1)

    return k


_sc_gather = _make_sc_gather()


@jax.jit
def kernel(x, table):
    idx = x.astype(jnp.int32).reshape(NW, NCHUNK, CHUNK)
    out = _sc_gather(table, idx)
    return out.reshape(BATCH, HIST, EMBED_DIM)


# speedup vs baseline: 1.8742x; 1.0714x over previous
import functools

import jax
import jax.numpy as jnp
from jax import lax
from jax.experimental import pallas as pl
from jax.experimental.pallas import tpu as pltpu
from jax.experimental.pallas import tpu_sc as plsc

BATCH = 16384
HIST = 50
EMBED_DIM = 64
N = BATCH * HIST
NC, NS = 2, 16
NW = NC * NS
PER_W = N // NW
CHUNK = 128
NCHUNK = PER_W // CHUNK
K = 4
GROUP = K * CHUNK
NGROUP = NCHUNK // K


def _make_sc_gather():
    mesh = plsc.VectorSubcoreMesh(
        core_axis_name="c", subcore_axis_name="s", num_cores=NC, num_subcores=NS
    )

    @functools.partial(
        pl.kernel,
        out_type=jax.ShapeDtypeStruct((N, EMBED_DIM), jnp.float32),
        mesh=mesh,
        compiler_params=pltpu.CompilerParams(use_tc_tiling_on_sc=False),
        scratch_types=[
            pltpu.VMEM((NCHUNK, CHUNK), jnp.int32),
            pltpu.VMEM((GROUP, EMBED_DIM), jnp.float32),
            pltpu.VMEM((GROUP, EMBED_DIM), jnp.float32),
            pltpu.SemaphoreType.DMA,
            pltpu.SemaphoreType.DMA,
            pltpu.SemaphoreType.DMA,
        ],
    )
    def k(table_hbm, idx_hbm, out_hbm, idx_v, buf_a, buf_b, gsem_a, gsem_b, ssem):
        wid = lax.axis_index("s") * NC + lax.axis_index("c")
        base = wid * PER_W
        pltpu.sync_copy(idx_hbm.at[wid], idx_v)

        def gather_group(g, buf, gsem):
            for b in range(K):
                pltpu.async_copy(
                    table_hbm.at[idx_v.at[g * K + b]],
                    buf.at[pl.ds(b * CHUNK, CHUNK)],
                    gsem,
                )

        def wait_gathers(buf, gsem):
            pltpu.make_async_copy(out_hbm.at[pl.ds(base, GROUP)], buf, gsem).wait()

        def put_group(g, buf):
            pltpu.async_copy(buf, out_hbm.at[pl.ds(base + g * GROUP, GROUP)], ssem)

        def drain_put(buf):
            pltpu.make_async_copy(buf, out_hbm.at[pl.ds(base, GROUP)], ssem).wait()

        gather_group(0, buf_a, gsem_a)
        wait_gathers(buf_a, gsem_a)
        gather_group(1, buf_b, gsem_b)
        put_group(0, buf_a)

        def body(i, carry):
            g1 = 2 * i + 1
            wait_gathers(buf_b, gsem_b)
            drain_put(buf_a)
            gather_group(g1 + 1, buf_a, gsem_a)
            put_group(g1, buf_b)
            g2 = 2 * i + 2
            wait_gathers(buf_a, gsem_a)
            drain_put(buf_b)
            gather_group(g2 + 1, buf_b, gsem_b)
            put_group(g2, buf_a)
            return carry

        lax.fori_loop(0, (NGROUP - 2) // 2, body, 0)
        wait_gathers(buf_b, gsem_b)
        drain_put(buf_a)
        put_group(NGROUP - 1, buf_b)
        drain_put(buf_b)

    return k


_sc_gather = _make_sc_gather()


@jax.jit
def kernel(x, table):
    idx = x.astype(jnp.int32).reshape(NW, NCHUNK, CHUNK)
    out = _sc_gather(table, idx)
    return out.reshape(BATCH, HIST, EMBED_DIM)
